# Initial kernel scaffold; baseline (speedup 1.0000x reference)
import jax
import jax.numpy as jnp
from jax.experimental import pallas as pl


def kernel(x, weight):
    raise NotImplementedError("write your pallas kernel here")


# speedup vs baseline: 5.0354x; 5.0354x over previous
import jax
import jax.numpy as jnp
from jax.experimental import pallas as pl


def _bcast_copy_kernel(w_ref, o_ref):
    o_ref[...] = jnp.broadcast_to(w_ref[...][None], o_ref.shape)


def kernel(x, weight):
    B, S = x.shape
    M, D = weight.shape
    BS = 512
    return pl.pallas_call(
        _bcast_copy_kernel,
        grid=(S // BS,),
        in_specs=[pl.BlockSpec((BS, D), lambda s: (s, 0))],
        out_specs=pl.BlockSpec((B, BS, D), lambda s: (0, s, 0)),
        out_shape=jax.ShapeDtypeStruct((B, S, D), weight.dtype),
    )(weight)


# speedup vs baseline: 5.1727x; 1.0273x over previous
import jax
import jax.numpy as jnp
from jax.experimental import pallas as pl


def _bcast_copy_kernel(w_ref, o_ref):
    o_ref[...] = jnp.broadcast_to(w_ref[...][None], o_ref.shape)


def kernel(x, weight):
    B, S = x.shape
    M, D = weight.shape
    BS = 1024
    return pl.pallas_call(
        _bcast_copy_kernel,
        grid=(S // BS,),
        in_specs=[pl.BlockSpec((BS, D), lambda s: (s, 0))],
        out_specs=pl.BlockSpec((B, BS, D), lambda s: (0, s, 0)),
        out_shape=jax.ShapeDtypeStruct((B, S, D), weight.dtype),
    )(weight)
